# baseline (speedup 1.0000x reference)
import functools

import jax
import jax.numpy as jnp
from jax import lax
from jax.experimental import pallas as pl
from jax.experimental.pallas import tpu as pltpu
from jax.experimental.pallas import tpu_sc as plsc

N = 10000
D = 128
E = 320000

NC = 2
NS = 16
NW = NC * NS

CH = 128
NCH = 80
EPW = NCH * CH
EPW_R = E // NW
N_PAD = 10112
RPW = N_PAD // NS


def _sc_agg_body(feat_hbm, idx_hbm, zeros_hbm, out_hbm,
                 acc_sh, idx_v, rows_v, sem):
    c = lax.axis_index("c")
    s = lax.axis_index("s")
    wid = s * NC + c

    pltpu.sync_copy(zeros_hbm.at[pl.ds(s * RPW, RPW)],
                    acc_sh.at[pl.ds(s * RPW, RPW)])
    plsc.subcore_barrier()

    gbase = wid * NCH

    def body(i, carry):
        pltpu.sync_copy(idx_hbm.at[gbase + i], idx_v)
        pltpu.make_async_copy(feat_hbm.at[idx_v.at[0]], rows_v, sem).start()
        pltpu.make_async_copy(feat_hbm.at[idx_v.at[0]], rows_v, sem).wait()
        pltpu.sync_copy(rows_v, acc_sh.at[idx_v.at[1]], add=True)
        return carry

    lax.fori_loop(0, NCH, body, 0)

    plsc.subcore_barrier()
    pltpu.sync_copy(acc_sh.at[pl.ds(s * RPW, RPW)],
                    out_hbm.at[c, pl.ds(s * RPW, RPW)])


def _sc_aggregate(feat, idx3, zeros):
    mesh = plsc.VectorSubcoreMesh(core_axis_name="c", subcore_axis_name="s")
    k = functools.partial(
        pl.kernel,
        mesh=mesh,
        out_type=jax.ShapeDtypeStruct((NC, N_PAD, D), jnp.float32),
        scratch_types=[
            pltpu.VMEM_SHARED((N_PAD, D), jnp.float32),
            pltpu.VMEM((2, CH), jnp.int32),
            pltpu.VMEM((CH, D), jnp.float32),
            pltpu.SemaphoreType.DMA,
        ],
    )(_sc_agg_body)
    return k(feat, idx3, zeros)


def _tc_combine_body(agg_ref, feat_ref, wn_ref, ws_ref, b_ref, out_ref):
    agg = agg_ref[0] + agg_ref[1]
    out_ref[...] = (
        jnp.dot(agg, wn_ref[...], preferred_element_type=jnp.float32)
        + jnp.dot(feat_ref[...], ws_ref[...], preferred_element_type=jnp.float32)
        + b_ref[...]
    )


def _tc_combine(agg2, feat, W_neigh, b_neigh, W_self):
    BR = 1000
    grid = N // BR
    return pl.pallas_call(
        _tc_combine_body,
        grid=(grid,),
        in_specs=[
            pl.BlockSpec((NC, BR, D), lambda i: (0, i, 0)),
            pl.BlockSpec((BR, D), lambda i: (i, 0)),
            pl.BlockSpec((D, D), lambda i: (0, 0)),
            pl.BlockSpec((D, D), lambda i: (0, 0)),
            pl.BlockSpec((1, D), lambda i: (0, 0)),
        ],
        out_specs=pl.BlockSpec((BR, D), lambda i: (i, 0)),
        out_shape=jax.ShapeDtypeStruct((N, D), jnp.float32),
    )(agg2, feat, W_neigh, W_self, b_neigh.reshape(1, D))


@jax.jit
def kernel(feat, edge_index, W_neigh, b_neigh, W_self):
    src = edge_index[0].astype(jnp.int32)
    dst = edge_index[1].astype(jnp.int32)
    pad = EPW - EPW_R
    src_p = jnp.pad(src.reshape(NW, EPW_R), ((0, 0), (0, pad)))
    dst_p = jnp.pad(dst.reshape(NW, EPW_R), ((0, 0), (0, pad)),
                    constant_values=N)
    idx3 = jnp.stack([src_p.reshape(NW * NCH, CH),
                      dst_p.reshape(NW * NCH, CH)], axis=1)
    zeros = jnp.zeros((N_PAD, D), jnp.float32)
    agg2 = _sc_aggregate(feat, idx3, zeros)
    return _tc_combine(agg2, feat, W_neigh, b_neigh, W_self)


# speedup vs baseline: 1.8954x; 1.8954x over previous
import functools

import jax
import jax.numpy as jnp
from jax import lax
from jax.experimental import pallas as pl
from jax.experimental.pallas import tpu as pltpu
from jax.experimental.pallas import tpu_sc as plsc

N = 10000
D = 128
E = 320000

NC = 2
NS = 16
NW = NC * NS

CH = 128
EPW = E // NW
NFULL = EPW // CH
TAIL = EPW - NFULL * CH
N_PAD = 10112
RPW = N_PAD // NS


def _sc_agg_body(feat_hbm, src_hbm, dst_hbm, zeros_hbm, out_hbm,
                 acc_sh, src_v, dst_v, rows_v, src_t, dst_t, rows_t, sem):
    c = lax.axis_index("c")
    s = lax.axis_index("s")
    wid = s * NC + c

    pltpu.sync_copy(zeros_hbm.at[pl.ds(s * RPW, RPW)],
                    acc_sh.at[pl.ds(s * RPW, RPW)])
    plsc.subcore_barrier()

    ebase = wid * EPW

    def body(i, carry):
        base = ebase + i * CH
        pltpu.sync_copy(src_hbm.at[pl.ds(base, CH)], src_v)
        pltpu.sync_copy(dst_hbm.at[pl.ds(base, CH)], dst_v)
        pltpu.make_async_copy(feat_hbm.at[src_v], rows_v, sem).start()
        pltpu.make_async_copy(feat_hbm.at[src_v], rows_v, sem).wait()
        pltpu.sync_copy(rows_v, acc_sh.at[dst_v], add=True)
        return carry

    lax.fori_loop(0, NFULL, body, 0)

    tbase = ebase + NFULL * CH
    pltpu.sync_copy(src_hbm.at[pl.ds(tbase, TAIL)], src_t)
    pltpu.sync_copy(dst_hbm.at[pl.ds(tbase, TAIL)], dst_t)
    pltpu.make_async_copy(feat_hbm.at[src_t], rows_t, sem).start()
    pltpu.make_async_copy(feat_hbm.at[src_t], rows_t, sem).wait()
    pltpu.sync_copy(rows_t, acc_sh.at[dst_t], add=True)

    plsc.subcore_barrier()
    pltpu.sync_copy(acc_sh.at[pl.ds(s * RPW, RPW)],
                    out_hbm.at[c, pl.ds(s * RPW, RPW)])


def _sc_aggregate(feat, src, dst, zeros):
    mesh = plsc.VectorSubcoreMesh(core_axis_name="c", subcore_axis_name="s")
    k = functools.partial(
        pl.kernel,
        mesh=mesh,
        out_type=jax.ShapeDtypeStruct((NC, N_PAD, D), jnp.float32),
        scratch_types=[
            pltpu.VMEM_SHARED((N_PAD, D), jnp.float32),
            pltpu.VMEM((CH,), jnp.int32),
            pltpu.VMEM((CH,), jnp.int32),
            pltpu.VMEM((CH, D), jnp.float32),
            pltpu.VMEM((TAIL,), jnp.int32),
            pltpu.VMEM((TAIL,), jnp.int32),
            pltpu.VMEM((TAIL, D), jnp.float32),
            pltpu.SemaphoreType.DMA,
        ],
    )(_sc_agg_body)
    return k(feat, src, dst, zeros)


def _tc_combine_body(agg_ref, feat_ref, wn_ref, ws_ref, b_ref, out_ref):
    agg = agg_ref[0] + agg_ref[1]
    out_ref[...] = (
        jnp.dot(agg, wn_ref[...], preferred_element_type=jnp.float32)
        + jnp.dot(feat_ref[...], ws_ref[...], preferred_element_type=jnp.float32)
        + b_ref[...]
    )


def _tc_combine(agg2, feat, W_neigh, b_neigh, W_self):
    BR = 1000
    grid = N // BR
    return pl.pallas_call(
        _tc_combine_body,
        grid=(grid,),
        in_specs=[
            pl.BlockSpec((NC, BR, D), lambda i: (0, i, 0)),
            pl.BlockSpec((BR, D), lambda i: (i, 0)),
            pl.BlockSpec((D, D), lambda i: (0, 0)),
            pl.BlockSpec((D, D), lambda i: (0, 0)),
            pl.BlockSpec((1, D), lambda i: (0, 0)),
        ],
        out_specs=pl.BlockSpec((BR, D), lambda i: (i, 0)),
        out_shape=jax.ShapeDtypeStruct((N, D), jnp.float32),
    )(agg2, feat, W_neigh, W_self, b_neigh.reshape(1, D))


@jax.jit
def kernel(feat, edge_index, W_neigh, b_neigh, W_self):
    src = edge_index[0].astype(jnp.int32)
    dst = edge_index[1].astype(jnp.int32)
    zeros = jnp.zeros((N_PAD, D), jnp.float32)
    agg2 = _sc_aggregate(feat, src, dst, zeros)
    return _tc_combine(agg2, feat, W_neigh, b_neigh, W_self)
